# baseline (speedup 1.0000x reference)
import jax
import jax.numpy as jnp
from jax.experimental import pallas as pl
from jax.experimental.pallas import tpu as pltpu

_K = 16
_DEPTH = 5
_LANES = 128
_RB = 64


def _dot(a, bt):
    return jax.lax.dot_general(
        a, bt, (((1,), (0,)), ((), ())),
        preferred_element_type=jnp.float32,
        precision=jax.lax.Precision.DEFAULT,
    )


def _exact_topk(rows, xt, sq_rows, sq_all, out_ref):
    g = _dot(rows, xt)
    d = sq_rows + (-2.0 * g) + sq_all
    n = d.shape[1]
    col = jax.lax.broadcasted_iota(jnp.int32, d.shape, 1)
    big = jnp.int32(n)
    cols = []
    for _ in range(_K):
        m = jnp.min(d, axis=1, keepdims=True)
        cand = jnp.where(d <= m, col, big)
        idx = jnp.min(cand, axis=1, keepdims=True)
        cols.append(idx)
        d = jnp.where(col == idx, jnp.float32(jnp.inf), d)
    out_ref[0] = jnp.concatenate(cols, axis=1)


def _knn_body(x_ref, xt_ref, out_ref, d_ref):
    rows = x_ref[0]
    xt = xt_ref[0]
    r = rows.shape[0]
    n = xt.shape[1]
    nchunks = n // _LANES
    inf = jnp.float32(jnp.inf)
    bigf = jnp.float32(n)

    sq_rows = jnp.sum(rows * rows, axis=1, keepdims=True)
    sq_all = jnp.sum(xt * xt, axis=0, keepdims=True)
    g = _dot(rows, xt)
    d_ref[...] = sq_rows + (-2.0 * g) + sq_all

    lane_f = jax.lax.broadcasted_iota(
        jnp.int32, (_RB, _LANES), 1).astype(jnp.float32)

    def block(i, flag):
        mv, iv = [], []
        for j in range(nchunks):
            v = d_ref[pl.ds(i * _RB, _RB), pl.ds(j * _LANES, _LANES)]
            c = lane_f + jnp.float32(j * _LANES)
            depth = len(mv)
            lt = [v < mv[k] for k in range(depth)]
            if depth < _DEPTH:
                mv.append(jnp.where(lt[-1], mv[-1], v) if depth else v)
                if depth < _DEPTH - 1:
                    iv.append(jnp.where(lt[-1], iv[-1], c) if depth else c)
            for k in range(depth - 1, 0, -1):
                mv[k] = jnp.where(lt[k - 1], mv[k - 1],
                                  jnp.where(lt[k], v, mv[k]))
                if k < _DEPTH - 1:
                    iv[k] = jnp.where(lt[k - 1], iv[k - 1],
                                      jnp.where(lt[k], c, iv[k]))
            if depth:
                mv[0] = jnp.where(lt[0], v, mv[0])
                iv[0] = jnp.where(lt[0], c, iv[0])

        cols = []
        exhausted = jnp.zeros((_RB, _LANES), jnp.bool_)
        for t in range(_K):
            m = jnp.min(mv[0], axis=1, keepdims=True)
            cand = jnp.where(mv[0] == m, iv[0], bigf)
            idx = jnp.min(cand, axis=1, keepdims=True)
            cols.append(idx)
            sel = cand == idx
            if t < _K - 1:
                exhausted = exhausted | (sel & (mv[1] == inf))
                for k in range(_DEPTH - 1):
                    mv[k] = jnp.where(sel, mv[k + 1], mv[k])
                    if k < _DEPTH - 2:
                        iv[k] = jnp.where(sel, iv[k + 1], iv[k])
                mv[_DEPTH - 1] = jnp.where(sel, inf, mv[_DEPTH - 1])
                iv[_DEPTH - 2] = jnp.where(sel, bigf, iv[_DEPTH - 2])
        res = jnp.concatenate(cols, axis=1).astype(jnp.int32)
        out_ref[0, pl.ds(i * _RB, _RB), :] = res
        return flag | jnp.any(exhausted)

    flag = jax.lax.fori_loop(0, r // _RB, block, False)

    @pl.when(flag)
    def _():
        _exact_topk(rows, xt, sq_rows, sq_all, out_ref)


def kernel(coords):
    b, n, dim = coords.shape
    xt = jnp.swapaxes(coords, 1, 2)
    r = 512
    nn_idx = pl.pallas_call(
        _knn_body,
        grid=(b, n // r),
        in_specs=[
            pl.BlockSpec((1, r, dim), lambda bi, i: (bi, i, 0)),
            pl.BlockSpec((1, dim, n), lambda bi, i: (bi, 0, 0)),
        ],
        out_specs=pl.BlockSpec((1, r, _K), lambda bi, i: (bi, i, 0)),
        out_shape=jax.ShapeDtypeStruct((b, n, _K), jnp.int32),
        scratch_shapes=[pltpu.VMEM((r, n), jnp.float32)],
    )(coords, xt)
    center = jnp.broadcast_to(
        jnp.arange(n, dtype=jnp.int32)[None, :, None], (b, n, _K)
    )
    return jnp.stack((nn_idx, center), axis=0)


# speedup vs baseline: 2.4476x; 2.4476x over previous
import jax
import jax.numpy as jnp
from jax.experimental import pallas as pl

_K = 16
_DEPTH = 5
_LANES = 128


def _dot(a, bt):
    return jax.lax.dot_general(
        a, bt, (((1,), (0,)), ((), ())),
        preferred_element_type=jnp.float32,
        precision=jax.lax.Precision.DEFAULT,
    )


def _exact_topk(rows, xt, sq_rows, sq_all, out_ref):
    g = _dot(rows, xt)
    d = sq_rows + (-2.0 * g) + sq_all
    n = d.shape[1]
    col = jax.lax.broadcasted_iota(jnp.int32, d.shape, 1)
    big = jnp.int32(n)
    cols = []
    for _ in range(_K):
        m = jnp.min(d, axis=1, keepdims=True)
        cand = jnp.where(d <= m, col, big)
        idx = jnp.min(cand, axis=1, keepdims=True)
        cols.append(idx)
        d = jnp.where(col == idx, jnp.float32(jnp.inf), d)
    out_ref[0] = jnp.concatenate(cols, axis=1)


def _knn_body(x_ref, xt_ref, out_ref):
    rows = x_ref[0]
    xt = xt_ref[0]
    r = rows.shape[0]
    n = xt.shape[1]
    nchunks = n // _LANES
    inf = jnp.float32(jnp.inf)
    bigf = jnp.float32(n)

    sq_rows = jnp.sum(rows * rows, axis=1, keepdims=True)
    sq_all = jnp.sum(xt * xt, axis=0, keepdims=True)

    mv, iv = [], []
    lane_f = jax.lax.broadcasted_iota(
        jnp.int32, (r, _LANES), 1).astype(jnp.float32)
    gall = _dot(rows, xt)
    for j in range(nchunks):
        lo = j * _LANES
        g = jax.lax.slice(gall, (0, lo), (r, lo + _LANES))
        sq_j = jax.lax.slice(sq_all, (0, lo), (1, lo + _LANES))
        v = sq_rows + (-2.0 * g) + sq_j
        c = lane_f + jnp.float32(lo)
        depth = len(mv)
        lt = [v < mv[k] for k in range(depth)]
        if depth < _DEPTH:
            mv.append(jnp.where(lt[-1], mv[-1], v) if depth else v)
            if depth < _DEPTH - 1:
                iv.append(jnp.where(lt[-1], iv[-1], c) if depth else c)
        for k in range(depth - 1, 0, -1):
            mv[k] = jnp.where(lt[k - 1], mv[k - 1],
                              jnp.where(lt[k], v, mv[k]))
            if k < _DEPTH - 1:
                iv[k] = jnp.where(lt[k - 1], iv[k - 1],
                                  jnp.where(lt[k], c, iv[k]))
        if depth:
            mv[0] = jnp.where(lt[0], v, mv[0])
            iv[0] = jnp.where(lt[0], c, iv[0])

    cols = []
    exhausted = jnp.zeros((r, _LANES), jnp.bool_)
    for t in range(_K):
        m = jnp.min(mv[0], axis=1, keepdims=True)
        cand = jnp.where(mv[0] == m, iv[0], bigf)
        idx = jnp.min(cand, axis=1, keepdims=True)
        cols.append(idx)
        sel = cand == idx
        if t < _K - 1:
            exhausted = exhausted | (sel & (mv[1] == inf))
            for k in range(_DEPTH - 1):
                mv[k] = jnp.where(sel, mv[k + 1], mv[k])
                if k < _DEPTH - 2:
                    iv[k] = jnp.where(sel, iv[k + 1], iv[k])
            mv[_DEPTH - 1] = jnp.where(sel, inf, mv[_DEPTH - 1])
            iv[_DEPTH - 2] = jnp.where(sel, bigf, iv[_DEPTH - 2])
    out_ref[0] = jnp.concatenate(cols, axis=1).astype(jnp.int32)

    @pl.when(jnp.any(exhausted))
    def _():
        _exact_topk(rows, xt, sq_rows, sq_all, out_ref)


def kernel(coords):
    b, n, dim = coords.shape
    xt = jnp.swapaxes(coords, 1, 2)
    r = 512
    nn_idx = pl.pallas_call(
        _knn_body,
        grid=(b, n // r),
        in_specs=[
            pl.BlockSpec((1, r, dim), lambda bi, i: (bi, i, 0)),
            pl.BlockSpec((1, dim, n), lambda bi, i: (bi, 0, 0)),
        ],
        out_specs=pl.BlockSpec((1, r, _K), lambda bi, i: (bi, i, 0)),
        out_shape=jax.ShapeDtypeStruct((b, n, _K), jnp.int32),
    )(coords, xt)
    center = jnp.broadcast_to(
        jnp.arange(n, dtype=jnp.int32)[None, :, None], (b, n, _K)
    )
    return jnp.stack((nn_idx, center), axis=0)
